# baseline (speedup 1.0000x reference)
import functools

import jax
import jax.numpy as jnp
from jax import lax
from jax.experimental import pallas as pl
from jax.experimental.pallas import tpu as pltpu
from jax.experimental.pallas import tpu_sc as plsc

B = 16384
F = 64
_NC = 2
_NS = 16
_NW = _NC * _NS
_BPW = B // _NW
_CHUNK = 128
_NCHUNK = _BPW // _CHUNK


def _sc_gather_body(uid_hbm, iid_hbm, pt_hbm, qt_hbm, ut_hbm, vt_hbm,
                    pm_hbm, qm_hbm, um_hbm, vm_hbm,
                    uid_v, iid_v, rp, rq, ru, rv, sp, sq, su, sv):
    wid = lax.axis_index("s") * _NC + lax.axis_index("c")
    crow = wid * _NCHUNK
    pltpu.sync_copy(uid_hbm.at[pl.ds(crow, _NCHUNK)], uid_v)
    pltpu.sync_copy(iid_hbm.at[pl.ds(crow, _NCHUNK)], iid_v)
    for k in range(_NCHUNK):
        cp = pltpu.async_copy(pt_hbm.at[uid_v.at[k]], rp, sp)
        cq = pltpu.async_copy(qt_hbm.at[iid_v.at[k]], rq, sq)
        cu = pltpu.async_copy(ut_hbm.at[uid_v.at[k]], ru, su)
        cv = pltpu.async_copy(vt_hbm.at[iid_v.at[k]], rv, sv)
        base = (crow + k) * _CHUNK
        cp.wait()
        pltpu.sync_copy(rp, pm_hbm.at[pl.ds(base, _CHUNK)])
        cq.wait()
        pltpu.sync_copy(rq, qm_hbm.at[pl.ds(base, _CHUNK)])
        cu.wait()
        pltpu.sync_copy(ru, um_hbm.at[pl.ds(base, _CHUNK)])
        cv.wait()
        pltpu.sync_copy(rv, vm_hbm.at[pl.ds(base, _CHUNK)])


_sc_gather = functools.partial(
    pl.kernel,
    mesh=plsc.VectorSubcoreMesh(core_axis_name="c", subcore_axis_name="s"),
    compiler_params=pltpu.CompilerParams(use_tc_tiling_on_sc=False),
    out_type=[jax.ShapeDtypeStruct((B, F), jnp.float32)] * 4,
    scratch_types=[
        pltpu.VMEM((_NCHUNK, _CHUNK), jnp.int32),
        pltpu.VMEM((_NCHUNK, _CHUNK), jnp.int32),
        pltpu.VMEM((_CHUNK, F), jnp.float32),
        pltpu.VMEM((_CHUNK, F), jnp.float32),
        pltpu.VMEM((_CHUNK, F), jnp.float32),
        pltpu.VMEM((_CHUNK, F), jnp.float32),
        pltpu.SemaphoreType.DMA,
        pltpu.SemaphoreType.DMA,
        pltpu.SemaphoreType.DMA,
        pltpu.SemaphoreType.DMA,
    ],
)(_sc_gather_body)


_BM = 2048


def _mlp_body(pm, qm, um, vm, w1a, w1b, b1, w2, b2, w3, b3, wpg, wph, bp, out):
    f32 = jnp.float32
    h = (jnp.dot(um[...], w1a[...], preferred_element_type=f32)
         + jnp.dot(vm[...], w1b[...], preferred_element_type=f32) + b1[...])
    h = jnp.maximum(h, 0.0)
    h = jnp.dot(h, w2[...], preferred_element_type=f32) + b2[...]
    h = jnp.maximum(h, 0.0)
    h = jnp.dot(h, w3[...], preferred_element_type=f32) + b3[...]
    h = jnp.maximum(h, 0.0)
    g = pm[...] * qm[...]
    z = (jnp.dot(g, wpg[...], preferred_element_type=f32)
         + jnp.dot(h, wph[...], preferred_element_type=f32) + bp[...])
    out[...] = 1.0 / (1.0 + jnp.exp(-z))


def _mlp_call(pm, qm, um, vm, w1a, w1b, b1, w2, b2, w3, b3, wpg, wph, bp):
    bm_spec = pl.BlockSpec((_BM, F), lambda i: (i, 0))
    full = lambda r, c: pl.BlockSpec((r, c), lambda i: (0, 0))
    return pl.pallas_call(
        _mlp_body,
        grid=(B // _BM,),
        in_specs=[
            bm_spec, bm_spec, bm_spec, bm_spec,
            full(F, 128), full(F, 128), full(1, 128),
            full(128, 64), full(1, 64),
            full(64, 32), full(1, 32),
            full(F, 1), full(32, 1), full(1, 1),
        ],
        out_specs=pl.BlockSpec((_BM, 1), lambda i: (i, 0)),
        out_shape=jax.ShapeDtypeStruct((B, 1), jnp.float32),
    )(pm, qm, um, vm, w1a, w1b, b1, w2, b2, w3, b3, wpg, wph, bp)


def kernel(user_id, item_id, Pt, Qt, Ut, Vt, W1, b1, W2, b2, W3, b3, Wp, bp):
    uid = user_id.astype(jnp.int32).reshape(B // _CHUNK, _CHUNK)
    iid = item_id.astype(jnp.int32).reshape(B // _CHUNK, _CHUNK)
    pm, qm, um, vm = _sc_gather(uid, iid, Pt, Qt, Ut, Vt)
    return _mlp_call(
        pm, qm, um, vm,
        W1[:F], W1[F:], b1.reshape(1, 128),
        W2, b2.reshape(1, 64),
        W3, b3.reshape(1, 32),
        Wp[:F], Wp[F:], bp.reshape(1, 1),
    )
